# baseline (speedup 1.0000x reference)
import functools

import jax
import jax.numpy as jnp
from jax import lax
from jax.experimental import pallas as pl
from jax.experimental.pallas import tpu as pltpu
from jax.experimental.pallas import tpu_sc as plsc

_LANES = 16
_CL = 8
_NW = 32


def kernel(x, add_table, mult_table):
    B, L, D = x.shape
    RW = L // _NW
    JC = RW // _CL
    mesh = plsc.VectorSubcoreMesh(core_axis_name="core",
                                  subcore_axis_name="subcore")

    def vm():
        return pltpu.VMEM((_CL, D), jnp.float32)

    scratch = (
        [vm() for _ in range(2 * B)]
        + [vm() for _ in range(4)]
        + [pltpu.SemaphoreType.DMA] * (2 * B)
        + [pltpu.SemaphoreType.DMA] * 4
        + [pltpu.SemaphoreType.DMA] * (2 * B)
    )

    @functools.partial(
        pl.kernel,
        out_type=jax.ShapeDtypeStruct((B, L, D), x.dtype),
        mesh=mesh,
        scratch_types=scratch,
    )
    def run(x_hbm, add_hbm, mult_hbm, o_hbm, *s):
        xb = s[0:2 * B]
        tb = s[2 * B:2 * B + 4]
        sx = s[2 * B + 4:4 * B + 4]
        st = s[4 * B + 4:4 * B + 8]
        so = s[4 * B + 8:6 * B + 8]
        wid = lax.axis_index("subcore") * 2 + lax.axis_index("core")
        l0 = wid * RW

        def start_in(j, ss):
            ls = pl.ds(l0 + j * _CL, _CL)
            cs = [pltpu.async_copy(x_hbm.at[b, ls], xb[ss * B + b],
                                   sx[ss * B + b]) for b in range(B)]
            cs.append(pltpu.async_copy(add_hbm.at[ls], tb[ss * 2 + 0],
                                       st[ss * 2 + 0]))
            cs.append(pltpu.async_copy(mult_hbm.at[ls], tb[ss * 2 + 1],
                                       st[ss * 2 + 1]))
            return cs

        def start_out(j, ss):
            ls = pl.ds(l0 + j * _CL, _CL)
            return [pltpu.async_copy(xb[ss * B + b], o_hbm.at[b, ls],
                                     so[ss * B + b]) for b in range(B)]

        ins = {0: start_in(0, 0)}
        outs = {}
        for j in range(JC):
            ss = j % 2
            if j + 1 < JC:
                if j - 1 >= 0:
                    for c in outs[j - 1]:
                        c.wait()
                ins[j + 1] = start_in(j + 1, (j + 1) % 2)
            for c in ins[j]:
                c.wait()

            @pl.loop(0, _CL)
            def _row(r, ss=ss):
                @pl.loop(0, D, step=_LANES, unroll=2)
                def _col(c, r=r, ss=ss):
                    sl = pl.ds(c, _LANES)
                    a = tb[ss * 2 + 0][r, sl]
                    m = tb[ss * 2 + 1][r, sl]
                    for b in range(B):
                        xb[ss * B + b][r, sl] = xb[ss * B + b][r, sl] * m + a

            outs[j] = start_out(j, ss)
        for j in (JC - 2, JC - 1):
            if j >= 0:
                for c in outs[j]:
                    c.wait()

    return run(x, add_table, mult_table)
